# Initial kernel scaffold; baseline (speedup 1.0000x reference)
import jax
import jax.numpy as jnp
from jax.experimental import pallas as pl


def kernel(x, x_time, W, b, month_tab, day_tab, weekday_tab, holiday_tab, hour_tab, event_tab, rain_tab):
    raise NotImplementedError("write your pallas kernel here")


# speedup vs baseline: 4.0058x; 4.0058x over previous
import jax
import jax.numpy as jnp
import numpy as np
from jax import lax
from jax.experimental import pallas as pl

_B, _T, _C, _D = 4, 4096, 32, 768
_MAX_LEN = 5000
_NF = 7
_OFFSETS = (0, 13, 45, 52, 54, 78, 80)
_NROWS = 128
_TB = 512


def _pe_table():
    position = jnp.arange(_MAX_LEN, dtype=jnp.float32)[:, None]
    div_term = jnp.exp(
        jnp.arange(0, _D, 2, dtype=jnp.float32) * (-(jnp.log(10000.0) / _D)))
    pe = jnp.zeros((_MAX_LEN, _D), dtype=jnp.float32)
    pe = pe.at[:, 0::2].set(jnp.sin(position * div_term))
    pe = pe.at[:, 1::2].set(jnp.cos(position * div_term))
    return pe[:_T]


def _fused_body(x_ref, xt_ref, wt_ref, tab_ref, b_ref, pe_ref, out_ref):
    x_blk = x_ref[...]
    idx = xt_ref[...]
    col = lax.broadcasted_iota(jnp.int32, (_TB, _NROWS), 1)
    oh = jnp.zeros((_TB, _NROWS), jnp.float32)
    for i, off in enumerate(_OFFSETS):
        oh += (col == idx[:, i][:, None] + off).astype(jnp.float32)
    acc = jnp.dot(x_blk, wt_ref[...], preferred_element_type=jnp.float32)
    acc += jnp.dot(oh, tab_ref[...], preferred_element_type=jnp.float32)
    out_ref[...] = acc + pe_ref[...] + b_ref[...]


def kernel(x, x_time, W, b, month_tab, day_tab, weekday_tab, holiday_tab,
           hour_tab, event_tab, rain_tab):
    n_tok = _B * _T
    xf = x.reshape(n_tok, _C)
    xt = x_time.reshape(n_tok, _NF)
    wt = W.T
    tab = jnp.concatenate(
        [month_tab, day_tab, weekday_tab, holiday_tab, hour_tab,
         event_tab, rain_tab], axis=0)
    tab = jnp.pad(tab, ((0, _NROWS - tab.shape[0]), (0, 0)))
    pe = _pe_table()
    n_blk = n_tok // _TB
    pe_blocks = _T // _TB

    out = pl.pallas_call(
        _fused_body,
        grid=(n_blk,),
        in_specs=[
            pl.BlockSpec((_TB, _C), lambda i: (i, 0)),
            pl.BlockSpec((_TB, _NF), lambda i: (i, 0)),
            pl.BlockSpec((_C, _D), lambda i: (0, 0)),
            pl.BlockSpec((_NROWS, _D), lambda i: (0, 0)),
            pl.BlockSpec((1, _D), lambda i: (0, 0)),
            pl.BlockSpec((_TB, _D), lambda i: (i % pe_blocks, 0)),
        ],
        out_specs=pl.BlockSpec((_TB, _D), lambda i: (i, 0)),
        out_shape=jax.ShapeDtypeStruct((n_tok, _D), jnp.float32),
    )(xf, xt, wt, tab, b.reshape(1, _D), pe)
    return out.reshape(_B, _T, _D)


# speedup vs baseline: 4.2679x; 1.0654x over previous
import jax
import jax.numpy as jnp
import numpy as np
from jax import lax
from jax.experimental import pallas as pl

_B, _T, _C, _D = 4, 4096, 32, 768
_MAX_LEN = 5000
_NF = 7
_OFFSETS = (0, 13, 45, 52, 54, 78, 80)
_NROWS = 128
_TB = 1024


def _pe_table():
    position = jnp.arange(_MAX_LEN, dtype=jnp.float32)[:, None]
    div_term = jnp.exp(
        jnp.arange(0, _D, 2, dtype=jnp.float32) * (-(jnp.log(10000.0) / _D)))
    pe = jnp.zeros((_MAX_LEN, _D), dtype=jnp.float32)
    pe = pe.at[:, 0::2].set(jnp.sin(position * div_term))
    pe = pe.at[:, 1::2].set(jnp.cos(position * div_term))
    return pe[:_T]


def _fused_body(x_ref, xt_ref, wt_ref, tab_ref, b_ref, pe_ref, out_ref):
    x_blk = x_ref[...]
    idx = xt_ref[...]
    col = lax.broadcasted_iota(jnp.int32, (_TB, _NROWS), 1)
    oh = jnp.zeros((_TB, _NROWS), jnp.float32)
    for i, off in enumerate(_OFFSETS):
        oh += (col == idx[:, i][:, None] + off).astype(jnp.float32)
    acc = jnp.dot(x_blk, wt_ref[...], preferred_element_type=jnp.float32)
    acc += jnp.dot(oh, tab_ref[...], preferred_element_type=jnp.float32)
    out_ref[...] = acc + pe_ref[...] + b_ref[...]


def kernel(x, x_time, W, b, month_tab, day_tab, weekday_tab, holiday_tab,
           hour_tab, event_tab, rain_tab):
    n_tok = _B * _T
    xf = x.reshape(n_tok, _C)
    xt = x_time.reshape(n_tok, _NF)
    wt = W.T
    tab = jnp.concatenate(
        [month_tab, day_tab, weekday_tab, holiday_tab, hour_tab,
         event_tab, rain_tab], axis=0)
    tab = jnp.pad(tab, ((0, _NROWS - tab.shape[0]), (0, 0)))
    pe = _pe_table()
    n_blk = n_tok // _TB
    pe_blocks = _T // _TB

    out = pl.pallas_call(
        _fused_body,
        grid=(n_blk,),
        in_specs=[
            pl.BlockSpec((_TB, _C), lambda i: (i, 0)),
            pl.BlockSpec((_TB, _NF), lambda i: (i, 0)),
            pl.BlockSpec((_C, _D), lambda i: (0, 0)),
            pl.BlockSpec((_NROWS, _D), lambda i: (0, 0)),
            pl.BlockSpec((1, _D), lambda i: (0, 0)),
            pl.BlockSpec((_TB, _D), lambda i: (i % pe_blocks, 0)),
        ],
        out_specs=pl.BlockSpec((_TB, _D), lambda i: (i, 0)),
        out_shape=jax.ShapeDtypeStruct((n_tok, _D), jnp.float32),
    )(xf, xt, wt, tab, b.reshape(1, _D), pe)
    return out.reshape(_B, _T, _D)
